# baseline (speedup 1.0000x reference)
import jax
import jax.numpy as jnp
from jax.experimental import pallas as pl
from jax.experimental.pallas import tpu as pltpu

E = 8
T = 2048
D = 2048
H = 1408
TM = 256
NT = T // TM
C13 = 2 * H // NT
C2 = D // NT


def _fused_body(x_ref, w13_hbm, w2_hbm, o_ref,
                w13a, w13b, w2a, w2b, stage13, stage2, sem13, sem2):
    e = pl.program_id(0)
    t = pl.program_id(1)
    cur = jax.lax.rem(e, 2)
    nxt = jax.lax.rem(e + 1, 2)

    def copies(src_e, c, slot):
        cp13 = pltpu.make_async_copy(
            w13_hbm.at[src_e, pl.ds(c * C13, C13), :],
            stage13.at[slot], sem13.at[slot])
        cp2 = pltpu.make_async_copy(
            w2_hbm.at[src_e, pl.ds(c * C2, C2), :],
            stage2.at[slot], sem2.at[slot])
        return cp13, cp2

    def start(src_e, c, slot):
        cp13, cp2 = copies(src_e, c, slot)
        cp13.start()
        cp2.start()

    def wait_cast(src_e, c, slot, ring13, ring2):
        cp13, cp2 = copies(src_e, c, slot)
        cp13.wait()
        cp2.wait()
        ring13[pl.ds(c * C13, C13), :] = stage13[slot].astype(jnp.bfloat16)
        ring2[pl.ds(c * C2, C2), :] = stage2[slot].astype(jnp.bfloat16)

    def maintenance(ring13, ring2, into_cur):
        if into_cur:
            @pl.when((e > 0) & (t == 0))
            def _():
                wait_cast(e, NT - 1, (NT - 1) % 2, ring13, ring2)
        else:
            @pl.when(e < E - 1)
            def _():
                start(e + 1, t, jax.lax.rem(t, 2))

            @pl.when((e < E - 1) & (t >= 1))
            def _():
                wait_cast(e + 1, t - 1, jax.lax.rem(t - 1, 2),
                          ring13, ring2)

    def compute(ring13, ring2):
        xb = x_ref[...].astype(jnp.bfloat16)
        h13 = jax.lax.dot_general(
            xb, ring13[...], (((1,), (1,)), ((), ())),
            preferred_element_type=jnp.float32)
        g = h13[:, :H]
        u = h13[:, H:]
        hb = ((g * jax.nn.sigmoid(g)) * u).astype(jnp.bfloat16)
        o_ref[...] = jax.lax.dot_general(
            hb, ring2[...], (((1,), (1,)), ((), ())),
            preferred_element_type=jnp.float32)

    @pl.when((e == 0) & (t == 0))
    def _():
        start(0, 0, 0)
        for c in range(NT):
            if c + 1 < NT:
                start(0, c + 1, (c + 1) % 2)
            wait_cast(0, c, c % 2, w13a, w2a)

    @pl.when(cur == 0)
    def _():
        maintenance(w13a, w2a, True)
        maintenance(w13b, w2b, False)
        compute(w13a, w2a)

    @pl.when(cur == 1)
    def _():
        maintenance(w13b, w2b, True)
        maintenance(w13a, w2a, False)
        compute(w13b, w2b)


def kernel(x, w13, w2, num_tokens_per_expert):
    out = pl.pallas_call(
        _fused_body,
        grid=(E, NT),
        in_specs=[
            pl.BlockSpec((TM, D), lambda e, t: (e * NT + t, 0)),
            pl.BlockSpec(memory_space=pl.ANY),
            pl.BlockSpec(memory_space=pl.ANY),
        ],
        out_specs=pl.BlockSpec((TM, D), lambda e, t: (e * NT + t, 0)),
        out_shape=jax.ShapeDtypeStruct((E * T, D), jnp.float32),
        scratch_shapes=[
            pltpu.VMEM((2 * H, D), jnp.bfloat16),
            pltpu.VMEM((2 * H, D), jnp.bfloat16),
            pltpu.VMEM((D, H), jnp.bfloat16),
            pltpu.VMEM((D, H), jnp.bfloat16),
            pltpu.VMEM((2, C13, D), jnp.float32),
            pltpu.VMEM((2, C2, H), jnp.float32),
            pltpu.SemaphoreType.DMA((2,)),
            pltpu.SemaphoreType.DMA((2,)),
        ],
        compiler_params=pltpu.CompilerParams(
            dimension_semantics=("parallel", "arbitrary")),
    )(x, w13, w2)
    return out
